# Initial kernel scaffold; baseline (speedup 1.0000x reference)
import jax
import jax.numpy as jnp
from jax.experimental import pallas as pl


def kernel(x, edge_index, batch, W_gcn, b_gcn, pool_w, gate_W, gate_b, W1, b1, W2, b2, Wc, bc):
    raise NotImplementedError("write your pallas kernel here")


# speedup vs baseline: 10.8499x; 10.8499x over previous
import functools

import jax
import jax.numpy as jnp
from jax import lax
from jax.experimental import pallas as pl
from jax.experimental.pallas import tpu as pltpu
from jax.experimental.pallas import tpu_sc as plsc

F32 = jnp.float32
I32 = jnp.int32

NGRAPH = 64
RATIO_F = 0.8

NC = 2
NS = 16
NW = NC * NS


def _worker_id():
    return lax.axis_index("s") * NC + lax.axis_index("c")


def _deg_body(n_pad, ep, dst_hbm, zeros_hbm, out_hbm, dstb, degb):
    w = _worker_id()
    ones16 = jnp.ones((16,), F32)
    pltpu.sync_copy(dst_hbm.at[w], dstb)
    pltpu.sync_copy(zeros_hbm, degb)

    def body(j, carry):
        for k in range(8):
            d16 = dstb[j, pl.ds(k * 16, 16)]
            plsc.addupdate_scatter(degb, [d16], ones16)
        return carry

    lax.fori_loop(0, ep, body, 0)
    pltpu.sync_copy(degb, out_hbm.at[w])


def _deg_call(dst3, zeros_n, n_pad, ep):
    mesh = plsc.VectorSubcoreMesh(core_axis_name="c", subcore_axis_name="s")
    f = pl.kernel(
        functools.partial(_deg_body, n_pad, ep),
        out_type=jax.ShapeDtypeStruct((NW, n_pad), F32),
        mesh=mesh,
        scratch_types=[
            pltpu.VMEM((ep, 128), I32),
            pltpu.VMEM((n_pad,), F32),
        ],
        compiler_params=pltpu.CompilerParams(needs_layout_passes=False),
    )
    return f(dst3, zeros_n)


def _edge_body(n_pad, nch, y_hbm, src_hbm, dst_hbm, zeros_hbm, out_hbm,
               ybuf, srcb, dstb, accb):
    w = _worker_id()
    pltpu.sync_copy(y_hbm.at[w], ybuf)
    for r in range(4):
        pltpu.sync_copy(zeros_hbm, accb.at[r])
    rvecs = [jnp.full((16,), r, I32) for r in range(4)]

    def chunk(ch, carry):
        pltpu.sync_copy(src_hbm.at[ch], srcb)
        pltpu.sync_copy(dst_hbm.at[ch], dstb)

        def row(j, c2):
            for k in range(8):
                s16 = srcb[j, pl.ds(k * 16, 16)]
                d16 = dstb[j, pl.ds(k * 16, 16)]
                for r in range(4):
                    v = plsc.load_gather(ybuf, [rvecs[r], s16])
                    plsc.addupdate_scatter(accb, [rvecs[r], d16], v)
            return c2

        lax.fori_loop(0, srcb.shape[0], row, 0)
        return carry

    lax.fori_loop(0, nch, chunk, 0)
    pltpu.sync_copy(accb, out_hbm.at[w])


def _scatter_call(yt, src3, dst3, zeros_n, n_pad):
    nch = src3.shape[0]
    mesh = plsc.VectorSubcoreMesh(core_axis_name="c", subcore_axis_name="s")
    f = pl.kernel(
        functools.partial(_edge_body, n_pad, nch),
        out_type=jax.ShapeDtypeStruct((NW, 4, n_pad), F32),
        mesh=mesh,
        scratch_types=[
            pltpu.VMEM((4, n_pad), F32),
            pltpu.VMEM((src3.shape[1], 128), I32),
            pltpu.VMEM((src3.shape[1], 128), I32),
            pltpu.VMEM((4, n_pad), F32),
        ],
        compiler_params=pltpu.CompilerParams(needs_layout_passes=False),
    )
    return f(yt, src3, dst3, zeros_n)


def _mid_body(n, x_ref, w_ref, deg_ref, y_ref):
    deg = jnp.sum(deg_ref[...], axis=0)[:n] + 1.0
    dis = lax.rsqrt(deg)
    xw = jnp.dot(x_ref[...], w_ref[...], preferred_element_type=F32)
    y_ref[...] = dis[:, None] * xw


def _mid_call(x, w_gcn, deg2):
    n = x.shape[0]
    return pl.pallas_call(
        functools.partial(_mid_body, n),
        out_shape=jax.ShapeDtypeStruct((n, 128), F32),
    )(x, w_gcn, deg2)


def _ha_body(n, acc_ref, y_ref, deg_ref, bgcn_ref, h_ref):
    deg = jnp.sum(deg_ref[...], axis=0)[:n] + 1.0
    dis = lax.rsqrt(deg)[:, None]
    acc = acc_ref[...] + y_ref[...]
    h_ref[...] = jnp.maximum(dis * acc + bgcn_ref[...], 0.0)


def _ha_call(acc2, y, deg2, b_gcn):
    n = y.shape[0]
    return pl.pallas_call(
        functools.partial(_ha_body, n),
        out_shape=jax.ShapeDtypeStruct((n, 128), F32),
    )(acc2, y, deg2, b_gcn)


def _post_body(n, np_, t, h_ref, batch_ref,
               poolw_ref, gatew_ref, gateb_ref, w1_ref, b1_ref, w2_ref,
               b2_ref, wc_ref, bc_ref, out_ref, score_s, batch_s):
    nt = np_ // t
    h = h_ref[...]
    hp = jnp.concatenate([h, jnp.zeros((np_ - n, 128), F32)], axis=0)

    poolw = poolw_ref[...]
    pnorm = jnp.sqrt(jnp.sum(poolw * poolw))
    score = jnp.sum(hp * poolw, axis=1, keepdims=True) / pnorm
    hg = jnp.sum(hp * gatew_ref[...], axis=1, keepdims=True)
    tsc = jnp.tanh(score)
    gate = tsc * hg + gateb_ref[0, 0]

    batch1 = batch_ref[...]
    gids = lax.broadcasted_iota(I32, (1, NGRAPH), 1)
    onehotb = batch1[:, None] == gids
    onehotf = onehotb.astype(F32)
    counts = jnp.sum(onehotf, axis=0)
    rix = lax.broadcasted_iota(I32, (NGRAPH, NGRAPH), 0)
    cix = lax.broadcasted_iota(I32, (NGRAPH, NGRAPH), 1)
    ltri = (cix < rix).astype(F32)
    starts = jnp.sum(ltri * counts[None, :], axis=1)
    ends = starts + counts
    kg = jnp.ceil(F32(RATIO_F) * counts)

    def sel(vec):
        return jnp.sum(jnp.where(onehotb, vec[None, :], 0.0), axis=1,
                       keepdims=True)

    wlo = sel(starts)
    whi = sel(ends)
    kq = sel(kg)

    score_s[...] = jnp.reshape(score[:, 0], (nt, t))
    batch_s[...] = jnp.reshape(batch1, (nt, t))
    iota_col = lax.broadcasted_iota(I32, (t, 1), 0)
    iota_row = lax.broadcasted_iota(I32, (1, t), 1)

    mask_tiles = []
    for it in range(nt):
        si = score[it * t:(it + 1) * t, :]
        bi = batch1[it * t:(it + 1) * t, None]
        gi = it * t + iota_col
        jlo = jnp.min(wlo[it * t:(it + 1) * t, :]).astype(I32) // t
        jhi = (jnp.max(whi[it * t:(it + 1) * t, :]).astype(I32) + (t - 1)) // t

        def jbody(jt, racc, si=si, bi=bi, gi=gi):
            sj = score_s[pl.ds(jt, 1), :]
            bj = batch_s[pl.ds(jt, 1), :]
            gj = jt * t + iota_row
            ahead = (sj > si) | ((sj == si) & (gj < gi))
            cnt = jnp.where((bj == bi) & ahead, 1.0, 0.0)
            return racc + jnp.sum(cnt, axis=1, keepdims=True)

        rank = lax.fori_loop(jlo, jhi, jbody, jnp.zeros((t, 1), F32))
        mask_tiles.append(rank < kq[it * t:(it + 1) * t, :])
    maskp = jnp.concatenate(mask_tiles, axis=0)

    wm = jnp.where(onehotb & maskp, gate, -1e30)
    gmax = jnp.max(wm, axis=0)
    gmax = jnp.where(gmax > -0.5e30, gmax, 0.0)
    e = jnp.where(maskp, jnp.exp(gate - sel(gmax)), 0.0)
    den = jnp.sum(onehotf * e, axis=0)
    den_n = sel(jnp.where(den > 0.0, den, 1.0))
    a = e / jnp.where(den_n > 0.0, den_n, 1.0)
    coef = a * tsc

    ge = lax.dot_general(onehotf, coef * hp,
                         (((0,), (0,)), ((), ())),
                         preferred_element_type=F32)
    z = jnp.maximum(jnp.dot(ge, w1_ref[...], preferred_element_type=F32)
                    + b1_ref[...], 0.0)
    z = jnp.maximum(jnp.dot(z, w2_ref[...], preferred_element_type=F32)
                    + b2_ref[...], 0.0)
    out_ref[...] = (jnp.dot(z, wc_ref[...], preferred_element_type=F32)
                    + bc_ref[...])


def _post_call(h, batch_p, pool_w, gate_w, gate_b,
               w1, b1, w2, b2, wc, bc, n, np_, t):
    return pl.pallas_call(
        functools.partial(_post_body, n, np_, t),
        out_shape=jax.ShapeDtypeStruct((NGRAPH, wc.shape[1]), F32),
        scratch_shapes=[pltpu.VMEM((np_ // t, t), F32),
                        pltpu.VMEM((np_ // t, t), I32)],
    )(h, batch_p, pool_w, gate_w, gate_b,
      w1, b1, w2, b2, wc, bc)


def kernel(x, edge_index, batch, W_gcn, b_gcn, pool_w, gate_W, gate_b,
           W1, b1, W2, b2, Wc, bc):
    n = x.shape[0]
    e = edge_index.shape[1]
    n_pad = -(-(n + 16) // 2048) * 2048
    ep = -(-e // (NW * 128))
    e_pad = NW * ep * 128

    src = jnp.concatenate([edge_index[0], jnp.zeros((e_pad - e,), I32)])
    dst = jnp.concatenate([edge_index[1], jnp.full((e_pad - e,), n, I32)])
    dst3 = dst.reshape(NW, ep, 128)
    cr = 32
    src_c = src.reshape(e_pad // (cr * 128), cr, 128)
    dst_c = dst.reshape(e_pad // (cr * 128), cr, 128)

    zeros_n = jnp.zeros((n_pad,), F32)

    deg32 = _deg_call(dst3, zeros_n, n_pad, ep)
    y = _mid_call(x, W_gcn, deg32)
    yt = jnp.concatenate([y, jnp.zeros((n_pad - n, 128), F32)]) \
        .T.reshape(NW, 4, n_pad)
    acc_t = _scatter_call(yt, src_c, dst_c, zeros_n, n_pad)
    acc = acc_t.reshape(128, n_pad).T[:n]

    t = 512
    np_ = -(-n // t) * t
    batch_p = jnp.concatenate([batch, jnp.full((np_ - n,), NGRAPH, I32)])
    h = _ha_call(acc, y, deg32, b_gcn.reshape(1, 128))
    out = _post_call(h, batch_p, pool_w.reshape(1, 128),
                     gate_W.reshape(1, 128), gate_b.reshape(1, 1),
                     W1, b1.reshape(1, -1), W2, b2.reshape(1, -1),
                     Wc, bc.reshape(1, -1), n, np_, t)
    return out
